# baseline (speedup 1.0000x reference)
import dataclasses

import jax
import jax.numpy as jnp
from jax import lax
from jax.experimental import pallas as pl
from jax.experimental.pallas import tpu as pltpu
from jax.experimental.pallas import tpu_sc as plsc

D = 128
LANES = 16
NSC = 2
NSUB = 16
NW = NSC * NSUB
CHUNK = 128

N1T = 5000
N2T = 1024

_SC_PARAMS = pltpu.CompilerParams()
if "needs_layout_passes" in pltpu.CompilerParams.__dataclass_fields__:
  _SC_PARAMS = dataclasses.replace(_SC_PARAMS, needs_layout_passes=False)


def _make_sc_agg(n_pad, groups, rows_per_tile):
  assert rows_per_tile * NSUB == n_pad and groups % 2 == 0

  def body(tab_hbm, src_hbm, dst_hbm, acc_hbm, cnt_hbm,
           acc_sh, sidx0, sidx1, didx0, didx1, rows0, rows1, hist_v,
           gsem0, gsem1, asem0, asem1):
    c = lax.axis_index("c")
    s = lax.axis_index("s")
    wid = c * NSUB + s
    sidx = (sidx0, sidx1)
    didx = (didx0, didx1)
    rows = (rows0, rows1)
    gsem = (gsem0, gsem1)
    asem = (asem0, asem1)

    zrow = jnp.zeros((LANES,), jnp.float32)
    one = jnp.ones((LANES,), jnp.float32)

    @pl.loop(0, CHUNK)
    def _(i):
      @pl.loop(0, D // LANES)
      def _(j):
        rows0[i, pl.ds(j * LANES, LANES)] = zrow

    @pl.loop(0, n_pad // LANES)
    def _(i):
      hist_v[pl.ds(i * LANES, LANES)] = zrow

    base_r = s * rows_per_tile
    off = 0
    left = rows_per_tile
    while left > 0:
      n = min(left, CHUNK)
      pltpu.sync_copy(rows0.at[pl.ds(0, n)], acc_sh.at[pl.ds(base_r + off, n)])
      off += n
      left -= n

    plsc.subcore_barrier()

    ebase = wid * groups * CHUNK

    pltpu.sync_copy(src_hbm.at[pl.ds(ebase, CHUNK)], sidx0)
    pltpu.sync_copy(dst_hbm.at[pl.ds(ebase, CHUNK)], didx0)
    pltpu.async_copy(tab_hbm.at[sidx0], rows0, gsem0)

    @pl.loop(0, groups // 2)
    def _(t):
      for b in (0, 1):
        g = 2 * t + b
        nb = 1 - b

        pltpu.make_async_copy(tab_hbm.at[sidx[b]], rows[b], gsem[b]).wait()
        pltpu.async_copy(rows[b], acc_sh.at[didx[b]], asem[b], add=True)

        def _drain():
          pltpu.make_async_copy(rows[nb], acc_sh.at[didx[nb]], asem[nb]).wait()
        if b == 0:
          pl.when(t > 0)(_drain)
        else:
          _drain()

        @pl.when(g + 1 < groups)
        def _():
          nbase = ebase + (g + 1) * CHUNK
          pltpu.sync_copy(src_hbm.at[pl.ds(nbase, CHUNK)], sidx[nb])
          pltpu.sync_copy(dst_hbm.at[pl.ds(nbase, CHUNK)], didx[nb])
          pltpu.async_copy(tab_hbm.at[sidx[nb]], rows[nb], gsem[nb])

        @pl.loop(0, CHUNK // LANES)
        def _(j):
          dvec = didx[b][pl.ds(j * LANES, LANES)]
          plsc.addupdate_scatter(hist_v, [dvec], one)

    pltpu.make_async_copy(rows1, acc_sh.at[didx1], asem1).wait()

    plsc.subcore_barrier()

    pltpu.sync_copy(acc_sh.at[pl.ds(base_r, rows_per_tile)],
                    acc_hbm.at[c, pl.ds(base_r, rows_per_tile)])
    pltpu.sync_copy(hist_v, cnt_hbm.at[wid])

  return pl.kernel(
      body,
      out_type=[jax.ShapeDtypeStruct((NSC, n_pad, D), jnp.float32),
                jax.ShapeDtypeStruct((NW, n_pad), jnp.float32)],
      mesh=plsc.VectorSubcoreMesh(core_axis_name="c", subcore_axis_name="s",
                                  num_cores=NSC, num_subcores=NSUB),
      compiler_params=_SC_PARAMS,
      scratch_types=[
          pltpu.VMEM_SHARED((n_pad, D), jnp.float32),
          pltpu.VMEM((CHUNK,), jnp.int32),
          pltpu.VMEM((CHUNK,), jnp.int32),
          pltpu.VMEM((CHUNK,), jnp.int32),
          pltpu.VMEM((CHUNK,), jnp.int32),
          pltpu.VMEM((CHUNK, D), jnp.float32),
          pltpu.VMEM((CHUNK, D), jnp.float32),
          pltpu.VMEM((n_pad,), jnp.float32),
          pltpu.SemaphoreType.DMA,
          pltpu.SemaphoreType.DMA,
          pltpu.SemaphoreType.DMA,
          pltpu.SemaphoreType.DMA,
      ],
  )


_G1 = 80
_PAD_N1 = 5120
_AGG1 = _make_sc_agg(_PAD_N1, _G1, _PAD_N1 // NSUB)
_G2 = 40
_PAD_N2 = 1152
_AGG2 = _make_sc_agg(_PAD_N2, _G2, _PAD_N2 // NSUB)


def _dense(p, cnt, x_tgt, wlT, b, wrT, n_rows, blk, last):
  def body(p_ref, c_ref, x_ref, wl_ref, b_ref, wr_ref, o_ref):
    cnt_col = jnp.maximum(jnp.sum(c_ref[...], axis=0), 1.0)[:, None]
    mean = (p_ref[0] + p_ref[1]) / cnt_col
    z = (jnp.dot(mean, wl_ref[...], preferred_element_type=jnp.float32)
         + b_ref[...]
         + jnp.dot(x_ref[...], wr_ref[...], preferred_element_type=jnp.float32))
    if last:
      m = jnp.max(z, axis=-1, keepdims=True)
      lse = jnp.log(jnp.sum(jnp.exp(z - m), axis=-1, keepdims=True)) + m
      o_ref[...] = z - lse
    else:
      o_ref[...] = jnp.maximum(z, 0.0)

  return pl.pallas_call(
      body,
      grid=(n_rows // blk,),
      in_specs=[
          pl.BlockSpec((NSC, blk, D), lambda i: (0, i, 0)),
          pl.BlockSpec((NW, blk), lambda i: (0, i)),
          pl.BlockSpec((blk, D), lambda i: (i, 0)),
          pl.BlockSpec((D, D), lambda i: (0, 0)),
          pl.BlockSpec((1, D), lambda i: (0, 0)),
          pl.BlockSpec((D, D), lambda i: (0, 0)),
      ],
      out_specs=pl.BlockSpec((blk, D), lambda i: (i, 0)),
      out_shape=jax.ShapeDtypeStruct((n_rows, D), jnp.float32),
  )(p, cnt, x_tgt, wlT, b, wrT)


def _pad_edges(edge_index, n_pad_edges, sink):
  src = edge_index[0].astype(jnp.int32)
  dst = edge_index[1].astype(jnp.int32)
  pad = n_pad_edges - src.shape[0]
  src = jnp.concatenate([src, jnp.zeros((pad,), jnp.int32)])
  dst = jnp.concatenate([dst, jnp.full((pad,), sink, jnp.int32)])
  return src, dst


def kernel(x, edge_index1, edge_index2, n1, n2, W_l1, b_l1, W_r1, W_l2, b_l2, W_r2):
  src1, dst1 = _pad_edges(edge_index1, NW * CHUNK * _G1, N1T)
  src2, dst2 = _pad_edges(edge_index2, NW * CHUNK * _G2, N2T)

  acc1, cnt1 = _AGG1(x, src1, dst1)
  x_tgt = lax.dynamic_slice_in_dim(x, n1 - N1T, _PAD_N1, axis=0)
  h = _dense(acc1, cnt1, x_tgt, W_l1.T, b_l1.reshape(1, D), W_r1.T,
             _PAD_N1, 1280, last=False)

  acc2, cnt2 = _AGG2(h, src2, dst2)
  h_tgt = lax.dynamic_slice_in_dim(h, n2 - N2T, N2T, axis=0)
  return _dense(acc2, cnt2, h_tgt, W_l2.T, b_l2.reshape(1, D), W_r2.T,
                N2T, N2T, last=True)


# speedup vs baseline: 1.0431x; 1.0431x over previous
import dataclasses

import jax
import jax.numpy as jnp
from jax import lax
from jax.experimental import pallas as pl
from jax.experimental.pallas import tpu as pltpu
from jax.experimental.pallas import tpu_sc as plsc

D = 128
LANES = 16
NSC = 2
NSUB = 16
NW = NSC * NSUB
CHUNK = 128

N1T = 5000
N2T = 1024

_SC_PARAMS = pltpu.CompilerParams()
if "needs_layout_passes" in pltpu.CompilerParams.__dataclass_fields__:
  _SC_PARAMS = dataclasses.replace(_SC_PARAMS, needs_layout_passes=False)


def _make_sc_agg(n_pad, groups, rows_per_tile):
  assert rows_per_tile * NSUB == n_pad and groups % 2 == 0

  def body(tab_hbm, src_hbm, dst_hbm, acc_hbm, cnt_hbm,
           acc_sh, sidx0, sidx1, didx0, didx1, rows0, rows1, hist_v,
           gsem0, gsem1):
    c = lax.axis_index("c")
    s = lax.axis_index("s")
    wid = c * NSUB + s
    sidx = (sidx0, sidx1)
    didx = (didx0, didx1)
    rows = (rows0, rows1)
    gsem = (gsem0, gsem1)

    zrow = jnp.zeros((LANES,), jnp.float32)
    one = jnp.ones((LANES,), jnp.float32)

    @pl.loop(0, CHUNK)
    def _(i):
      @pl.loop(0, D // LANES)
      def _(j):
        rows0[i, pl.ds(j * LANES, LANES)] = zrow

    @pl.loop(0, n_pad // LANES)
    def _(i):
      hist_v[pl.ds(i * LANES, LANES)] = zrow

    base_r = s * rows_per_tile
    off = 0
    left = rows_per_tile
    while left > 0:
      n = min(left, CHUNK)
      pltpu.sync_copy(rows0.at[pl.ds(0, n)], acc_sh.at[pl.ds(base_r + off, n)])
      off += n
      left -= n

    ebase = wid * groups * CHUNK

    pltpu.async_copy(dst_hbm.at[pl.ds(ebase, CHUNK)], didx0, gsem0)

    @pl.loop(0, groups // 2)
    def _(t):
      for b in (0, 1):
        g = 2 * t + b
        nb = 1 - b
        pltpu.make_async_copy(dst_hbm.at[pl.ds(ebase, CHUNK)], didx[b],
                              gsem[b]).wait()

        @pl.when(g + 1 < groups)
        def _():
          nbase = ebase + (g + 1) * CHUNK
          pltpu.async_copy(dst_hbm.at[pl.ds(nbase, CHUNK)], didx[nb], gsem[nb])

        for j in range(CHUNK // LANES):
          dvec = didx[b][pl.ds(j * LANES, LANES)]
          plsc.addupdate_scatter(hist_v, [dvec], one)

    plsc.subcore_barrier()

    pltpu.sync_copy(src_hbm.at[pl.ds(ebase, CHUNK)], sidx0)
    pltpu.sync_copy(dst_hbm.at[pl.ds(ebase, CHUNK)], didx0)
    pltpu.async_copy(tab_hbm.at[sidx0], rows0, gsem0)

    @pl.loop(0, groups // 2)
    def _(t):
      for b in (0, 1):
        g = 2 * t + b
        nb = 1 - b

        @pl.when(g + 1 < groups)
        def _():
          nbase = ebase + (g + 1) * CHUNK
          pltpu.sync_copy(src_hbm.at[pl.ds(nbase, CHUNK)], sidx[nb])
          pltpu.sync_copy(dst_hbm.at[pl.ds(nbase, CHUNK)], didx[nb])
          pltpu.async_copy(tab_hbm.at[sidx[nb]], rows[nb], gsem[nb])

        pltpu.make_async_copy(tab_hbm.at[sidx[b]], rows[b], gsem[b]).wait()
        pltpu.sync_copy(rows[b], acc_sh.at[didx[b]], add=True)

    plsc.subcore_barrier()

    pltpu.sync_copy(acc_sh.at[pl.ds(base_r, rows_per_tile)],
                    acc_hbm.at[c, pl.ds(base_r, rows_per_tile)])
    pltpu.sync_copy(hist_v, cnt_hbm.at[wid])

  return pl.kernel(
      body,
      out_type=[jax.ShapeDtypeStruct((NSC, n_pad, D), jnp.float32),
                jax.ShapeDtypeStruct((NW, n_pad), jnp.float32)],
      mesh=plsc.VectorSubcoreMesh(core_axis_name="c", subcore_axis_name="s",
                                  num_cores=NSC, num_subcores=NSUB),
      compiler_params=_SC_PARAMS,
      scratch_types=[
          pltpu.VMEM_SHARED((n_pad, D), jnp.float32),
          pltpu.VMEM((CHUNK,), jnp.int32),
          pltpu.VMEM((CHUNK,), jnp.int32),
          pltpu.VMEM((CHUNK,), jnp.int32),
          pltpu.VMEM((CHUNK,), jnp.int32),
          pltpu.VMEM((CHUNK, D), jnp.float32),
          pltpu.VMEM((CHUNK, D), jnp.float32),
          pltpu.VMEM((n_pad,), jnp.float32),
          pltpu.SemaphoreType.DMA,
          pltpu.SemaphoreType.DMA,
      ],
  )


_G1 = 80
_PAD_N1 = 5120
_AGG1 = _make_sc_agg(_PAD_N1, _G1, _PAD_N1 // NSUB)
_G2 = 40
_PAD_N2 = 1152
_AGG2 = _make_sc_agg(_PAD_N2, _G2, _PAD_N2 // NSUB)


def _dense(p, cnt, x_tgt, wlT, b, wrT, n_rows, blk, last):
  def body(p_ref, c_ref, x_ref, wl_ref, b_ref, wr_ref, o_ref):
    cnt_col = jnp.maximum(jnp.sum(c_ref[...], axis=0), 1.0)[:, None]
    mean = (p_ref[0] + p_ref[1]) / cnt_col
    z = (jnp.dot(mean, wl_ref[...], preferred_element_type=jnp.float32)
         + b_ref[...]
         + jnp.dot(x_ref[...], wr_ref[...], preferred_element_type=jnp.float32))
    if last:
      m = jnp.max(z, axis=-1, keepdims=True)
      lse = jnp.log(jnp.sum(jnp.exp(z - m), axis=-1, keepdims=True)) + m
      o_ref[...] = z - lse
    else:
      o_ref[...] = jnp.maximum(z, 0.0)

  return pl.pallas_call(
      body,
      grid=(n_rows // blk,),
      in_specs=[
          pl.BlockSpec((NSC, blk, D), lambda i: (0, i, 0)),
          pl.BlockSpec((NW, blk), lambda i: (0, i)),
          pl.BlockSpec((blk, D), lambda i: (i, 0)),
          pl.BlockSpec((D, D), lambda i: (0, 0)),
          pl.BlockSpec((1, D), lambda i: (0, 0)),
          pl.BlockSpec((D, D), lambda i: (0, 0)),
      ],
      out_specs=pl.BlockSpec((blk, D), lambda i: (i, 0)),
      out_shape=jax.ShapeDtypeStruct((n_rows, D), jnp.float32),
  )(p, cnt, x_tgt, wlT, b, wrT)


def _pad_edges(edge_index, n_pad_edges, sink):
  src = edge_index[0].astype(jnp.int32)
  dst = edge_index[1].astype(jnp.int32)
  pad = n_pad_edges - src.shape[0]
  src = jnp.concatenate([src, jnp.zeros((pad,), jnp.int32)])
  dst = jnp.concatenate([dst, jnp.full((pad,), sink, jnp.int32)])
  return src, dst


def kernel(x, edge_index1, edge_index2, n1, n2, W_l1, b_l1, W_r1, W_l2, b_l2, W_r2):
  src1, dst1 = _pad_edges(edge_index1, NW * CHUNK * _G1, N1T)
  src2, dst2 = _pad_edges(edge_index2, NW * CHUNK * _G2, N2T)

  acc1, cnt1 = _AGG1(x, src1, dst1)
  x_tgt = lax.dynamic_slice_in_dim(x, n1 - N1T, _PAD_N1, axis=0)
  h = _dense(acc1, cnt1, x_tgt, W_l1.T, b_l1.reshape(1, D), W_r1.T,
             _PAD_N1, 1280, last=False)

  acc2, cnt2 = _AGG2(h, src2, dst2)
  h_tgt = lax.dynamic_slice_in_dim(h, n2 - N2T, N2T, axis=0)
  return _dense(acc2, cnt2, h_tgt, W_l2.T, b_l2.reshape(1, D), W_r2.T,
                N2T, N2T, last=True)


# speedup vs baseline: 1.3812x; 1.3241x over previous
import jax
import jax.numpy as jnp
from jax import lax
from jax.experimental import pallas as pl
from jax.experimental.pallas import tpu as pltpu
from jax.experimental.pallas import tpu_sc as plsc

D = 128
LANES = 16
NSC = 2
NSUB = 16
NW = NSC * NSUB
CHUNK = 112

N1T = 5000
N2T = 1024


def _make_sc_agg(n_pad, groups, rows_per_tile):
  assert rows_per_tile * NSUB == n_pad and groups % 2 == 0

  def body(tab_hbm, src_hbm, dst_hbm, acc_hbm, cnt_hbm,
           acc_sh, cnt_sh, sidx0, sidx1, didx0, didx1, rows0, rows1,
           ones_v, gsem0, gsem1):
    c = lax.axis_index("c")
    s = lax.axis_index("s")
    wid = c * NSUB + s
    sidx = (sidx0, sidx1)
    didx = (didx0, didx1)
    rows = (rows0, rows1)
    gsem = (gsem0, gsem1)

    zrow = jnp.zeros((LANES,), jnp.float32)
    one = jnp.ones((LANES,), jnp.float32)

    @pl.loop(0, CHUNK)
    def _(i):
      @pl.loop(0, D // LANES)
      def _(j):
        rows0[i, pl.ds(j * LANES, LANES)] = zrow
        ones_v[i, pl.ds(j * LANES, LANES)] = one

    base_r = s * rows_per_tile
    off = 0
    left = rows_per_tile
    while left > 0:
      n = min(left, CHUNK)
      pltpu.sync_copy(rows0.at[pl.ds(0, n)], acc_sh.at[pl.ds(base_r + off, n)])
      pltpu.sync_copy(rows0.at[pl.ds(0, n)], cnt_sh.at[pl.ds(base_r + off, n)])
      off += n
      left -= n

    plsc.subcore_barrier()

    ebase = wid * groups * CHUNK

    pltpu.sync_copy(src_hbm.at[pl.ds(ebase, CHUNK)], sidx0)
    pltpu.sync_copy(dst_hbm.at[pl.ds(ebase, CHUNK)], didx0)
    pltpu.async_copy(tab_hbm.at[sidx0], rows0, gsem0)

    @pl.loop(0, groups // 2)
    def _(t):
      for b in (0, 1):
        g = 2 * t + b
        nb = 1 - b

        @pl.when(g + 1 < groups)
        def _():
          nbase = ebase + (g + 1) * CHUNK
          pltpu.sync_copy(src_hbm.at[pl.ds(nbase, CHUNK)], sidx[nb])
          pltpu.sync_copy(dst_hbm.at[pl.ds(nbase, CHUNK)], didx[nb])
          pltpu.async_copy(tab_hbm.at[sidx[nb]], rows[nb], gsem[nb])

        pltpu.make_async_copy(tab_hbm.at[sidx[b]], rows[b], gsem[b]).wait()
        pltpu.sync_copy(rows[b], acc_sh.at[didx[b]], add=True)
        pltpu.sync_copy(ones_v, cnt_sh.at[didx[b]], add=True)

    plsc.subcore_barrier()

    pltpu.sync_copy(acc_sh.at[pl.ds(base_r, rows_per_tile)],
                    acc_hbm.at[c, pl.ds(base_r, rows_per_tile)])
    pltpu.sync_copy(cnt_sh.at[pl.ds(base_r, rows_per_tile)],
                    cnt_hbm.at[c, pl.ds(base_r, rows_per_tile)])

  return pl.kernel(
      body,
      out_type=[jax.ShapeDtypeStruct((NSC, n_pad, D), jnp.float32),
                jax.ShapeDtypeStruct((NSC, n_pad, D), jnp.float32)],
      mesh=plsc.VectorSubcoreMesh(core_axis_name="c", subcore_axis_name="s",
                                  num_cores=NSC, num_subcores=NSUB),
      scratch_types=[
          pltpu.VMEM_SHARED((n_pad, D), jnp.float32),
          pltpu.VMEM_SHARED((n_pad, D), jnp.float32),
          pltpu.VMEM((CHUNK,), jnp.int32),
          pltpu.VMEM((CHUNK,), jnp.int32),
          pltpu.VMEM((CHUNK,), jnp.int32),
          pltpu.VMEM((CHUNK,), jnp.int32),
          pltpu.VMEM((CHUNK, D), jnp.float32),
          pltpu.VMEM((CHUNK, D), jnp.float32),
          pltpu.VMEM((CHUNK, D), jnp.float32),
          pltpu.SemaphoreType.DMA,
          pltpu.SemaphoreType.DMA,
      ],
  )


_G1 = 90
_PAD_N1 = 5120
_AGG1 = _make_sc_agg(_PAD_N1, _G1, _PAD_N1 // NSUB)
_G2 = 46
_PAD_N2 = 1152
_AGG2 = _make_sc_agg(_PAD_N2, _G2, _PAD_N2 // NSUB)


def _dense(p, cnt, x_tgt, wlT, b, wrT, n_rows, blk, last):
  def body(p_ref, c_ref, x_ref, wl_ref, b_ref, wr_ref, o_ref):
    cnt_col = jnp.maximum(c_ref[0, :, 0:1] + c_ref[1, :, 0:1], 1.0)
    mean = (p_ref[0] + p_ref[1]) / cnt_col
    z = (jnp.dot(mean, wl_ref[...], preferred_element_type=jnp.float32)
         + b_ref[...]
         + jnp.dot(x_ref[...], wr_ref[...], preferred_element_type=jnp.float32))
    if last:
      m = jnp.max(z, axis=-1, keepdims=True)
      lse = jnp.log(jnp.sum(jnp.exp(z - m), axis=-1, keepdims=True)) + m
      o_ref[...] = z - lse
    else:
      o_ref[...] = jnp.maximum(z, 0.0)

  return pl.pallas_call(
      body,
      grid=(n_rows // blk,),
      in_specs=[
          pl.BlockSpec((NSC, blk, D), lambda i: (0, i, 0)),
          pl.BlockSpec((NSC, blk, D), lambda i: (0, i, 0)),
          pl.BlockSpec((blk, D), lambda i: (i, 0)),
          pl.BlockSpec((D, D), lambda i: (0, 0)),
          pl.BlockSpec((1, D), lambda i: (0, 0)),
          pl.BlockSpec((D, D), lambda i: (0, 0)),
      ],
      out_specs=pl.BlockSpec((blk, D), lambda i: (i, 0)),
      out_shape=jax.ShapeDtypeStruct((n_rows, D), jnp.float32),
  )(p, cnt, x_tgt, wlT, b, wrT)


def _pad_edges(edge_index, n_pad_edges, sink):
  src = edge_index[0].astype(jnp.int32)
  dst = edge_index[1].astype(jnp.int32)
  pad = n_pad_edges - src.shape[0]
  src = jnp.concatenate([src, jnp.zeros((pad,), jnp.int32)])
  dst = jnp.concatenate([dst, jnp.full((pad,), sink, jnp.int32)])
  return src, dst


def kernel(x, edge_index1, edge_index2, n1, n2, W_l1, b_l1, W_r1, W_l2, b_l2, W_r2):
  src1, dst1 = _pad_edges(edge_index1, NW * CHUNK * _G1, N1T)
  src2, dst2 = _pad_edges(edge_index2, NW * CHUNK * _G2, N2T)

  acc1, cnt1 = _AGG1(x, src1, dst1)
  x_tgt = lax.dynamic_slice_in_dim(x, n1 - N1T, N1T, axis=0)
  h = _dense(acc1, cnt1, x_tgt, W_l1.T, b_l1.reshape(1, D), W_r1.T,
             N1T, 1000, last=False)

  acc2, cnt2 = _AGG2(h, src2, dst2)
  h_tgt = lax.dynamic_slice_in_dim(h, n2 - N2T, N2T, axis=0)
  return _dense(acc2, cnt2, h_tgt, W_l2.T, b_l2.reshape(1, D), W_r2.T,
                N2T, N2T, last=True)


# speedup vs baseline: 2.4645x; 1.7844x over previous
import jax
import jax.numpy as jnp
from jax import lax
from jax.experimental import pallas as pl
from jax.experimental.pallas import tpu as pltpu
from jax.experimental.pallas import tpu_sc as plsc

D = 128
LANES = 16
NSC = 2
NSUB = 16
NW = NSC * NSUB
CHUNK = 112

N1T = 5000
N2T = 1024


def _make_sc_agg(n_pad, groups, rows_per_tile):
  assert rows_per_tile * NSUB == n_pad and groups % 2 == 0

  def body(tab_hbm, src_hbm, dst_hbm, acc_hbm, cnt_hbm,
           acc_sh, cnt_sh, sidx0, sidx1, didx0, didx1, rows0, rows1,
           ones_v, gsem0, gsem1):
    c = lax.axis_index("c")
    s = lax.axis_index("s")
    wid = c * NSUB + s
    sidx = (sidx0, sidx1)
    didx = (didx0, didx1)
    rows = (rows0, rows1)
    gsem = (gsem0, gsem1)

    zrow = jnp.zeros((LANES,), jnp.float32)
    one = jnp.ones((LANES,), jnp.float32)

    @pl.loop(0, CHUNK)
    def _(i):
      @pl.loop(0, D // LANES)
      def _(j):
        rows0[i, pl.ds(j * LANES, LANES)] = zrow
        ones_v[i, pl.ds(j * LANES, LANES)] = one

    base_r = s * rows_per_tile
    off = 0
    left = rows_per_tile
    while left > 0:
      n = min(left, CHUNK)
      pltpu.sync_copy(rows0.at[pl.ds(0, n)], acc_sh.at[pl.ds(base_r + off, n)])
      pltpu.sync_copy(rows0.at[pl.ds(0, n)], cnt_sh.at[pl.ds(base_r + off, n)])
      off += n
      left -= n

    plsc.subcore_barrier()

    ebase = wid * groups * CHUNK

    pltpu.sync_copy(src_hbm.at[pl.ds(ebase, CHUNK)], sidx0)
    pltpu.sync_copy(dst_hbm.at[pl.ds(ebase, CHUNK)], didx0)
    pltpu.async_copy(tab_hbm.at[sidx0], rows0, gsem0)

    @pl.loop(0, groups // 2)
    def _(t):
      for b in (0, 1):
        g = 2 * t + b
        nb = 1 - b

        @pl.when(g + 1 < groups)
        def _():
          nbase = ebase + (g + 1) * CHUNK
          pltpu.sync_copy(src_hbm.at[pl.ds(nbase, CHUNK)], sidx[nb])
          pltpu.sync_copy(dst_hbm.at[pl.ds(nbase, CHUNK)], didx[nb])
          pltpu.async_copy(tab_hbm.at[sidx[nb]], rows[nb], gsem[nb])

        pltpu.make_async_copy(tab_hbm.at[sidx[b]], rows[b], gsem[b]).wait()
        pltpu.sync_copy(rows[b], acc_sh.at[didx[b]], add=True)
        pltpu.sync_copy(ones_v, cnt_sh.at[didx[b]], add=True)

    plsc.subcore_barrier()

    pltpu.sync_copy(acc_sh.at[pl.ds(base_r, rows_per_tile)],
                    acc_hbm.at[c, pl.ds(base_r, rows_per_tile)])
    pltpu.sync_copy(cnt_sh.at[pl.ds(base_r, rows_per_tile)],
                    cnt_hbm.at[c, pl.ds(base_r, rows_per_tile)])

  return pl.kernel(
      body,
      out_type=[jax.ShapeDtypeStruct((NSC, n_pad, D), jnp.float32),
                jax.ShapeDtypeStruct((NSC, n_pad, D), jnp.float32)],
      mesh=plsc.VectorSubcoreMesh(core_axis_name="c", subcore_axis_name="s",
                                  num_cores=NSC, num_subcores=NSUB),
      scratch_types=[
          pltpu.VMEM_SHARED((n_pad, D), jnp.float32),
          pltpu.VMEM_SHARED((n_pad, D), jnp.float32),
          pltpu.VMEM((CHUNK,), jnp.int32),
          pltpu.VMEM((CHUNK,), jnp.int32),
          pltpu.VMEM((CHUNK,), jnp.int32),
          pltpu.VMEM((CHUNK,), jnp.int32),
          pltpu.VMEM((CHUNK, D), jnp.float32),
          pltpu.VMEM((CHUNK, D), jnp.float32),
          pltpu.VMEM((CHUNK, D), jnp.float32),
          pltpu.SemaphoreType.DMA,
          pltpu.SemaphoreType.DMA,
      ],
  )


_G1 = 90
_PAD_N1 = 5120
_AGG1 = _make_sc_agg(_PAD_N1, _G1, _PAD_N1 // NSUB)
_G2 = 46
_PAD_N2 = 1152
_AGG2 = _make_sc_agg(_PAD_N2, _G2, _PAD_N2 // NSUB)


def _dense(p, cnt, x_tgt, wlT, b, wrT, n_rows, blk, last):
  def body(p_ref, c_ref, x_ref, wl_ref, b_ref, wr_ref, o_ref):
    cnt_col = jnp.maximum(c_ref[0, :, 0:1] + c_ref[1, :, 0:1], 1.0)
    mean = (p_ref[0] + p_ref[1]) / cnt_col
    z = (jnp.dot(mean, wl_ref[...], preferred_element_type=jnp.float32)
         + b_ref[...]
         + jnp.dot(x_ref[...], wr_ref[...], preferred_element_type=jnp.float32))
    if last:
      m = jnp.max(z, axis=-1, keepdims=True)
      lse = jnp.log(jnp.sum(jnp.exp(z - m), axis=-1, keepdims=True)) + m
      o_ref[...] = z - lse
    else:
      o_ref[...] = jnp.maximum(z, 0.0)

  return pl.pallas_call(
      body,
      grid=(n_rows // blk,),
      in_specs=[
          pl.BlockSpec((NSC, blk, D), lambda i: (0, i, 0)),
          pl.BlockSpec((NSC, blk, D), lambda i: (0, i, 0)),
          pl.BlockSpec((blk, D), lambda i: (i, 0)),
          pl.BlockSpec((D, D), lambda i: (0, 0)),
          pl.BlockSpec((1, D), lambda i: (0, 0)),
          pl.BlockSpec((D, D), lambda i: (0, 0)),
      ],
      out_specs=pl.BlockSpec((blk, D), lambda i: (i, 0)),
      out_shape=jax.ShapeDtypeStruct((n_rows, D), jnp.float32),
  )(p, cnt, x_tgt, wlT, b, wrT)


def _pad_edges(edge_index, n_pad_edges, sink, n_sink):
  src = edge_index[0].astype(jnp.int32)
  dst = edge_index[1].astype(jnp.int32)
  pad = n_pad_edges - src.shape[0]
  i = jnp.arange(pad, dtype=jnp.int32)
  src = jnp.concatenate([src, i % sink])
  dst = jnp.concatenate([dst, sink + i % n_sink])
  return src, dst


def kernel(x, edge_index1, edge_index2, n1, n2, W_l1, b_l1, W_r1, W_l2, b_l2, W_r2):
  src1, dst1 = _pad_edges(edge_index1, NW * CHUNK * _G1, N1T, _PAD_N1 - N1T)
  src2, dst2 = _pad_edges(edge_index2, NW * CHUNK * _G2, N2T, _PAD_N2 - N2T)

  acc1, cnt1 = _AGG1(x, src1, dst1)
  x_tgt = lax.dynamic_slice_in_dim(x, n1 - N1T, N1T, axis=0)
  h = _dense(acc1, cnt1, x_tgt, W_l1.T, b_l1.reshape(1, D), W_r1.T,
             N1T, 1000, last=False)

  acc2, cnt2 = _AGG2(h, src2, dst2)
  h_tgt = lax.dynamic_slice_in_dim(h, n2 - N2T, N2T, axis=0)
  return _dense(acc2, cnt2, h_tgt, W_l2.T, b_l2.reshape(1, D), W_r2.T,
                N2T, N2T, last=True)
